# Initial kernel scaffold; baseline (speedup 1.0000x reference)
import jax
import jax.numpy as jnp
from jax.experimental import pallas as pl


def kernel(x, emb):
    raise NotImplementedError("write your pallas kernel here")


# speedup vs baseline: 1.0902x; 1.0902x over previous
import functools

import jax
import jax.numpy as jnp
from jax import lax
from jax.experimental import pallas as pl
from jax.experimental.pallas import tpu as pltpu
from jax.experimental.pallas import tpu_sc as plsc

N_CODES = 8192
EMB_D = 256
N_TOKENS = 8192
BR = 256
COMMIT = 0.25

NW = 32
ROWS_PER_W = N_TOKENS // NW
GCH = 128


def _dist_argmin_body(flat2_ref, srow_ref, e_ref, se_ref, idx_ref, dsum_ref):
    i = pl.program_id(0)
    flat2 = flat2_ref[...]
    e = e_ref[...]
    m2 = lax.dot_general(flat2, e, (((1,), (1,)), ((), ())),
                         preferred_element_type=jnp.float32)
    srow = srow_ref[...]
    se = se_ref[...]
    dist = (srow + se) - m2
    HC = N_CODES // 2
    d1 = dist[:, :HC]
    d2 = dist[:, HC:]
    v1 = jnp.min(d1, axis=1, keepdims=True)
    v2 = jnp.min(d2, axis=1, keepdims=True)
    iota = lax.broadcasted_iota(jnp.int32, (BR, HC), 1)
    i1 = jnp.min(jnp.where(d1 == v1, iota, N_CODES), axis=1)
    i2 = jnp.min(jnp.where(d2 == v2, iota + HC, N_CODES), axis=1)
    v1b = v1.astype(jnp.bfloat16).astype(jnp.float32)
    take1 = v1b <= v2
    idx = jnp.where(take1[:, 0], i1, i2)
    idx_ref[0, 0, :] = idx
    minval = jnp.where(take1, v1, v2)
    bsum = jnp.full((1, 128), jnp.sum(minval), jnp.float32)

    @pl.when(i == 0)
    def _():
        dsum_ref[...] = jnp.zeros((1, 128), jnp.float32)

    dsum_ref[...] += bsum


@functools.lru_cache(maxsize=1)
def _make_sc_gather():
    mesh = plsc.VectorSubcoreMesh(core_axis_name="c", subcore_axis_name="s",
                                  num_cores=2, num_subcores=16)

    @functools.partial(
        pl.kernel,
        out_type=jax.ShapeDtypeStruct((N_TOKENS, EMB_D), jnp.float32),
        mesh=mesh,
        scratch_types=[
            pltpu.VMEM((GCH,), jnp.int32),
            pltpu.VMEM((GCH,), jnp.int32),
            pltpu.VMEM((GCH, EMB_D), jnp.float32),
            pltpu.VMEM((GCH, EMB_D), jnp.float32),
            pltpu.SemaphoreType.DMA,
            pltpu.SemaphoreType.DMA,
        ],
    )
    def _sc_gather(table_hbm, idx_hbm, out_hbm, idx_v0, idx_v1,
                   rows_v0, rows_v1, sem0, sem1):
        wid = lax.axis_index("s") * 2 + lax.axis_index("c")
        base = wid * ROWS_PER_W
        idx_vs = (idx_v0, idx_v1)
        rows_vs = (rows_v0, rows_v1)
        sems = (sem0, sem1)
        n_ch = ROWS_PER_W // GCH
        for j in range(n_ch):
            pltpu.sync_copy(idx_hbm.at[pl.ds(base + j * GCH, GCH)],
                            idx_vs[j % 2])
            pltpu.async_copy(table_hbm.at[idx_vs[j % 2]], rows_vs[j % 2],
                             sems[j % 2])
        for j in range(n_ch):
            pltpu.make_async_copy(table_hbm.at[idx_vs[j % 2]], rows_vs[j % 2],
                                  sems[j % 2]).wait()
            pltpu.sync_copy(rows_vs[j % 2],
                            out_hbm.at[pl.ds(base + j * GCH, GCH)])

    return _sc_gather


def kernel(x, emb):
    B, D, T, H, W = x.shape
    x_dtype = x.dtype
    x32 = x.astype(jnp.float32)
    x_flat = jnp.transpose(x32, (0, 2, 3, 4, 1))
    flat = x_flat.reshape(-1, D)
    e = emb.astype(jnp.float32)
    srow = jnp.sum(flat ** 2, axis=1, keepdims=True)
    se = jnp.sum(e ** 2, axis=1)[None, :]
    flat2_bf = (2.0 * flat).astype(jnp.bfloat16)
    e_bf = e.astype(jnp.bfloat16)

    idx3, dsum = pl.pallas_call(
        _dist_argmin_body,
        grid=(N_TOKENS // BR,),
        in_specs=[
            pl.BlockSpec((BR, D), lambda i: (i, 0)),
            pl.BlockSpec((BR, 1), lambda i: (i, 0)),
            pl.BlockSpec((N_CODES, D), lambda i: (0, 0)),
            pl.BlockSpec((1, N_CODES), lambda i: (0, 0)),
        ],
        out_specs=[
            pl.BlockSpec((1, 1, BR), lambda i: (i, 0, 0)),
            pl.BlockSpec((1, 128), lambda i: (0, 0)),
        ],
        out_shape=[
            jax.ShapeDtypeStruct((N_TOKENS // BR, 1, BR), jnp.int32),
            jax.ShapeDtypeStruct((1, 128), jnp.float32),
        ],
    )(flat2_bf, srow, e_bf, se)

    indices = idx3.reshape(N_TOKENS)
    quantized = _make_sc_gather()(e_bf.astype(jnp.float32), indices)

    mse = dsum[0, 0] / (N_TOKENS * D)
    vq_loss = mse + COMMIT * mse

    q5 = quantized.reshape(B, T, H, W, D)
    quantized_st = x_flat + lax.stop_gradient(q5 - x_flat)
    quantized_st = jnp.transpose(quantized_st, (0, 4, 1, 2, 3)).astype(x_dtype)
    indices_out = indices.reshape(B, T, H, W)
    return quantized_st, vq_loss, indices_out


# speedup vs baseline: 1.2050x; 1.1052x over previous
import functools

import jax
import jax.numpy as jnp
from jax import lax
from jax.experimental import pallas as pl
from jax.experimental.pallas import tpu as pltpu
from jax.experimental.pallas import tpu_sc as plsc

N_CODES = 8192
EMB_D = 256
N_TOKENS = 8192
BR = 256
COMMIT = 0.25

NW = 32
ROWS_PER_W = N_TOKENS // NW
GCH = 128


CH = 1024


def _dist_argmin_body(flat2_ref, srow_ref, e_ref, se_ref, fio_ref,
                      idx_ref, dsum_ref):
    i = pl.program_id(0)
    flat2 = flat2_ref[...]
    srow = srow_ref[...]
    big = float(2 * N_CODES)
    HC = N_CODES // 2
    half_v, half_f = [], []
    for h in range(2):
        rv = rf = None
        for c in range(HC // CH):
            j0 = h * HC + c * CH
            e_c = e_ref[pl.ds(j0, CH), :]
            m2 = lax.dot_general(flat2, e_c, (((1,), (1,)), ((), ())),
                                 preferred_element_type=jnp.float32)
            dist = (srow + se_ref[:, pl.ds(j0, CH)]) - m2
            cv = jnp.min(dist, axis=1, keepdims=True)
            cf = jnp.min(jnp.where(dist == cv, fio_ref[:, pl.ds(j0, CH)],
                                   big), axis=1, keepdims=True)
            if rv is None:
                rv, rf = cv, cf
            else:
                t = cv < rv
                rv = jnp.where(t, cv, rv)
                rf = jnp.where(t, cf, rf)
        half_v.append(rv)
        half_f.append(rf)
    v1b = half_v[0].astype(jnp.bfloat16).astype(jnp.float32)
    take1 = v1b <= half_v[1]
    idx = jnp.where(take1[:, 0], half_f[0][:, 0],
                    half_f[1][:, 0]).astype(jnp.int32)
    idx_ref[0, 0, :] = idx
    minval = jnp.where(take1, half_v[0], half_v[1])
    bsum = jnp.full((1, 128), jnp.sum(minval), jnp.float32)

    @pl.when(i == 0)
    def _():
        dsum_ref[...] = jnp.zeros((1, 128), jnp.float32)

    dsum_ref[...] += bsum


@functools.lru_cache(maxsize=1)
def _make_sc_gather():
    mesh = plsc.VectorSubcoreMesh(core_axis_name="c", subcore_axis_name="s",
                                  num_cores=2, num_subcores=16)

    @functools.partial(
        pl.kernel,
        out_type=jax.ShapeDtypeStruct((N_TOKENS, EMB_D), jnp.float32),
        mesh=mesh,
        scratch_types=[
            pltpu.VMEM((GCH,), jnp.int32),
            pltpu.VMEM((GCH,), jnp.int32),
            pltpu.VMEM((GCH, EMB_D), jnp.float32),
            pltpu.VMEM((GCH, EMB_D), jnp.float32),
            pltpu.SemaphoreType.DMA,
            pltpu.SemaphoreType.DMA,
        ],
    )
    def _sc_gather(table_hbm, idx_hbm, out_hbm, idx_v0, idx_v1,
                   rows_v0, rows_v1, sem0, sem1):
        wid = lax.axis_index("s") * 2 + lax.axis_index("c")
        base = wid * ROWS_PER_W
        idx_vs = (idx_v0, idx_v1)
        rows_vs = (rows_v0, rows_v1)
        sems = (sem0, sem1)
        n_ch = ROWS_PER_W // GCH
        for j in range(n_ch):
            pltpu.sync_copy(idx_hbm.at[pl.ds(base + j * GCH, GCH)],
                            idx_vs[j % 2])
            pltpu.async_copy(table_hbm.at[idx_vs[j % 2]], rows_vs[j % 2],
                             sems[j % 2])
        for j in range(n_ch):
            pltpu.make_async_copy(table_hbm.at[idx_vs[j % 2]], rows_vs[j % 2],
                                  sems[j % 2]).wait()
            pltpu.sync_copy(rows_vs[j % 2],
                            out_hbm.at[pl.ds(base + j * GCH, GCH)])

    return _sc_gather


def kernel(x, emb):
    B, D, T, H, W = x.shape
    x_dtype = x.dtype
    x32 = x.astype(jnp.float32)
    x_flat = jnp.transpose(x32, (0, 2, 3, 4, 1))
    flat = x_flat.reshape(-1, D)
    e = emb.astype(jnp.float32)
    srow = jnp.sum(flat ** 2, axis=1, keepdims=True)
    se = jnp.sum(e ** 2, axis=1)[None, :]
    flat2_bf = (2.0 * flat).astype(jnp.bfloat16)
    e_bf = e.astype(jnp.bfloat16)

    idx3, dsum = pl.pallas_call(
        _dist_argmin_body,
        grid=(N_TOKENS // BR,),
        in_specs=[
            pl.BlockSpec((BR, D), lambda i: (i, 0)),
            pl.BlockSpec((BR, 1), lambda i: (i, 0)),
            pl.BlockSpec((N_CODES, D), lambda i: (0, 0)),
            pl.BlockSpec((1, N_CODES), lambda i: (0, 0)),
            pl.BlockSpec((1, N_CODES), lambda i: (0, 0)),
        ],
        out_specs=[
            pl.BlockSpec((1, 1, BR), lambda i: (i, 0, 0)),
            pl.BlockSpec((1, 128), lambda i: (0, 0)),
        ],
        out_shape=[
            jax.ShapeDtypeStruct((N_TOKENS // BR, 1, BR), jnp.int32),
            jax.ShapeDtypeStruct((1, 128), jnp.float32),
        ],
    )(flat2_bf, srow, e_bf, se,
      jnp.arange(N_CODES, dtype=jnp.float32)[None, :])

    indices = idx3.reshape(N_TOKENS)
    quantized = _make_sc_gather()(e_bf.astype(jnp.float32), indices)

    mse = dsum[0, 0] / (N_TOKENS * D)
    vq_loss = mse + COMMIT * mse

    q5 = quantized.reshape(B, T, H, W, D)
    quantized_st = x_flat + lax.stop_gradient(q5 - x_flat)
    quantized_st = jnp.transpose(quantized_st, (0, 4, 1, 2, 3)).astype(x_dtype)
    indices_out = indices.reshape(B, T, H, W)
    return quantized_st, vq_loss, indices_out


# speedup vs baseline: 1.2397x; 1.0288x over previous
import functools

import jax
import jax.numpy as jnp
from jax import lax
from jax.experimental import pallas as pl
from jax.experimental.pallas import tpu as pltpu
from jax.experimental.pallas import tpu_sc as plsc

N_CODES = 8192
EMB_D = 256
N_TOKENS = 8192
BR = 512
COMMIT = 0.25

NW = 32
ROWS_PER_W = N_TOKENS // NW
GCH = 128


CH = 1024


def _dist_argmin_body(flat2_ref, srow_ref, e_ref, se_ref, fio_ref,
                      idx_ref, dsum_ref):
    i = pl.program_id(0)
    flat2 = flat2_ref[...]
    srow = srow_ref[...]
    big = float(2 * N_CODES)
    HC = N_CODES // 2
    half_v, half_f = [], []
    for h in range(2):
        rv = rf = None
        for c in range(HC // CH):
            j0 = h * HC + c * CH
            e_c = e_ref[pl.ds(j0, CH), :]
            m2 = lax.dot_general(flat2, e_c, (((1,), (1,)), ((), ())),
                                 preferred_element_type=jnp.float32)
            dist = (srow + se_ref[:, pl.ds(j0, CH)]) - m2
            cv = jnp.min(dist, axis=1, keepdims=True)
            cf = jnp.min(jnp.where(dist == cv, fio_ref[:, pl.ds(j0, CH)],
                                   big), axis=1, keepdims=True)
            if rv is None:
                rv, rf = cv, cf
            else:
                t = cv < rv
                rv = jnp.where(t, cv, rv)
                rf = jnp.where(t, cf, rf)
        half_v.append(rv)
        half_f.append(rf)
    v1b = half_v[0].astype(jnp.bfloat16).astype(jnp.float32)
    take1 = v1b <= half_v[1]
    idx = jnp.where(take1[:, 0], half_f[0][:, 0],
                    half_f[1][:, 0]).astype(jnp.int32)
    idx_ref[0, 0, :] = idx
    minval = jnp.where(take1, half_v[0], half_v[1])
    bsum = jnp.full((1, 128), jnp.sum(minval), jnp.float32)

    @pl.when(i == 0)
    def _():
        dsum_ref[...] = jnp.zeros((1, 128), jnp.float32)

    dsum_ref[...] += bsum


@functools.lru_cache(maxsize=1)
def _make_sc_gather():
    mesh = plsc.VectorSubcoreMesh(core_axis_name="c", subcore_axis_name="s",
                                  num_cores=2, num_subcores=16)

    @functools.partial(
        pl.kernel,
        out_type=jax.ShapeDtypeStruct((N_TOKENS, EMB_D), jnp.float32),
        mesh=mesh,
        scratch_types=[
            pltpu.VMEM((GCH,), jnp.int32),
            pltpu.VMEM((GCH,), jnp.int32),
            pltpu.VMEM((GCH, EMB_D), jnp.float32),
            pltpu.VMEM((GCH, EMB_D), jnp.float32),
            pltpu.SemaphoreType.DMA,
            pltpu.SemaphoreType.DMA,
        ],
    )
    def _sc_gather(table_hbm, idx_hbm, out_hbm, idx_v0, idx_v1,
                   rows_v0, rows_v1, sem0, sem1):
        wid = lax.axis_index("s") * 2 + lax.axis_index("c")
        base = wid * ROWS_PER_W
        idx_vs = (idx_v0, idx_v1)
        rows_vs = (rows_v0, rows_v1)
        sems = (sem0, sem1)
        n_ch = ROWS_PER_W // GCH
        for j in range(n_ch):
            pltpu.sync_copy(idx_hbm.at[pl.ds(base + j * GCH, GCH)],
                            idx_vs[j % 2])
            pltpu.async_copy(table_hbm.at[idx_vs[j % 2]], rows_vs[j % 2],
                             sems[j % 2])
        for j in range(n_ch):
            pltpu.make_async_copy(table_hbm.at[idx_vs[j % 2]], rows_vs[j % 2],
                                  sems[j % 2]).wait()
            pltpu.sync_copy(rows_vs[j % 2],
                            out_hbm.at[pl.ds(base + j * GCH, GCH)])

    return _sc_gather


def kernel(x, emb):
    B, D, T, H, W = x.shape
    x_dtype = x.dtype
    x32 = x.astype(jnp.float32)
    x_flat = jnp.transpose(x32, (0, 2, 3, 4, 1))
    flat = x_flat.reshape(-1, D)
    e = emb.astype(jnp.float32)
    srow = jnp.sum(flat ** 2, axis=1, keepdims=True)
    se = jnp.sum(e ** 2, axis=1)[None, :]
    flat2_bf = (2.0 * flat).astype(jnp.bfloat16)
    e_bf = e.astype(jnp.bfloat16)

    idx3, dsum = pl.pallas_call(
        _dist_argmin_body,
        grid=(N_TOKENS // BR,),
        in_specs=[
            pl.BlockSpec((BR, D), lambda i: (i, 0)),
            pl.BlockSpec((BR, 1), lambda i: (i, 0)),
            pl.BlockSpec((N_CODES, D), lambda i: (0, 0)),
            pl.BlockSpec((1, N_CODES), lambda i: (0, 0)),
            pl.BlockSpec((1, N_CODES), lambda i: (0, 0)),
        ],
        out_specs=[
            pl.BlockSpec((1, 1, BR), lambda i: (i, 0, 0)),
            pl.BlockSpec((1, 128), lambda i: (0, 0)),
        ],
        out_shape=[
            jax.ShapeDtypeStruct((N_TOKENS // BR, 1, BR), jnp.int32),
            jax.ShapeDtypeStruct((1, 128), jnp.float32),
        ],
    )(flat2_bf, srow, e_bf, se,
      jnp.arange(N_CODES, dtype=jnp.float32)[None, :])

    indices = idx3.reshape(N_TOKENS)
    quantized = _make_sc_gather()(e_bf.astype(jnp.float32), indices)

    mse = dsum[0, 0] / (N_TOKENS * D)
    vq_loss = mse + COMMIT * mse

    q5 = quantized.reshape(B, T, H, W, D)
    quantized_st = x_flat + lax.stop_gradient(q5 - x_flat)
    quantized_st = jnp.transpose(quantized_st, (0, 4, 1, 2, 3)).astype(x_dtype)
    indices_out = indices.reshape(B, T, H, W)
    return quantized_st, vq_loss, indices_out


# speedup vs baseline: 1.2796x; 1.0321x over previous
import functools

import jax
import jax.numpy as jnp
from jax import lax
from jax.experimental import pallas as pl
from jax.experimental.pallas import tpu as pltpu
from jax.experimental.pallas import tpu_sc as plsc

N_CODES = 8192
EMB_D = 256
N_TOKENS = 8192
BR = 1024
COMMIT = 0.25

NW = 32
ROWS_PER_W = N_TOKENS // NW
GCH = 128


CH = 1024


def _dist_argmin_body(flat2_ref, srow_ref, e_ref, se_ref, fio_ref,
                      idx_ref, dsum_ref):
    i = pl.program_id(0)
    flat2 = flat2_ref[...]
    srow = srow_ref[...]
    big = float(2 * N_CODES)
    HC = N_CODES // 2
    half_v, half_f = [], []
    for h in range(2):
        rv = rf = None
        for c in range(HC // CH):
            j0 = h * HC + c * CH
            e_c = e_ref[pl.ds(j0, CH), :]
            m2 = lax.dot_general(flat2, e_c, (((1,), (1,)), ((), ())),
                                 preferred_element_type=jnp.float32)
            dist = (srow + se_ref[:, pl.ds(j0, CH)]) - m2
            cv = jnp.min(dist, axis=1, keepdims=True)
            cf = jnp.min(jnp.where(dist == cv, fio_ref[:, pl.ds(j0, CH)],
                                   big), axis=1, keepdims=True)
            if rv is None:
                rv, rf = cv, cf
            else:
                t = cv < rv
                rv = jnp.where(t, cv, rv)
                rf = jnp.where(t, cf, rf)
        half_v.append(rv)
        half_f.append(rf)
    v1b = half_v[0].astype(jnp.bfloat16).astype(jnp.float32)
    take1 = v1b <= half_v[1]
    idx = jnp.where(take1[:, 0], half_f[0][:, 0],
                    half_f[1][:, 0]).astype(jnp.int32)
    idx_ref[0, 0, :] = idx
    minval = jnp.where(take1, half_v[0], half_v[1])
    bsum = jnp.full((1, 128), jnp.sum(minval), jnp.float32)

    @pl.when(i == 0)
    def _():
        dsum_ref[...] = jnp.zeros((1, 128), jnp.float32)

    dsum_ref[...] += bsum


@functools.lru_cache(maxsize=1)
def _make_sc_gather():
    mesh = plsc.VectorSubcoreMesh(core_axis_name="c", subcore_axis_name="s",
                                  num_cores=2, num_subcores=16)

    @functools.partial(
        pl.kernel,
        out_type=jax.ShapeDtypeStruct((N_TOKENS, EMB_D), jnp.float32),
        mesh=mesh,
        scratch_types=[
            pltpu.VMEM((GCH,), jnp.int32),
            pltpu.VMEM((GCH,), jnp.int32),
            pltpu.VMEM((GCH, EMB_D), jnp.float32),
            pltpu.VMEM((GCH, EMB_D), jnp.float32),
            pltpu.SemaphoreType.DMA,
            pltpu.SemaphoreType.DMA,
        ],
    )
    def _sc_gather(table_hbm, idx_hbm, out_hbm, idx_v0, idx_v1,
                   rows_v0, rows_v1, sem0, sem1):
        wid = lax.axis_index("s") * 2 + lax.axis_index("c")
        base = wid * ROWS_PER_W
        idx_vs = (idx_v0, idx_v1)
        rows_vs = (rows_v0, rows_v1)
        sems = (sem0, sem1)
        n_ch = ROWS_PER_W // GCH
        for j in range(n_ch):
            pltpu.sync_copy(idx_hbm.at[pl.ds(base + j * GCH, GCH)],
                            idx_vs[j % 2])
            pltpu.async_copy(table_hbm.at[idx_vs[j % 2]], rows_vs[j % 2],
                             sems[j % 2])
        for j in range(n_ch):
            pltpu.make_async_copy(table_hbm.at[idx_vs[j % 2]], rows_vs[j % 2],
                                  sems[j % 2]).wait()
            pltpu.sync_copy(rows_vs[j % 2],
                            out_hbm.at[pl.ds(base + j * GCH, GCH)])

    return _sc_gather


def kernel(x, emb):
    B, D, T, H, W = x.shape
    x_dtype = x.dtype
    x32 = x.astype(jnp.float32)
    x_flat = jnp.transpose(x32, (0, 2, 3, 4, 1))
    flat = x_flat.reshape(-1, D)
    e = emb.astype(jnp.float32)
    srow = jnp.sum(flat ** 2, axis=1, keepdims=True)
    se = jnp.sum(e ** 2, axis=1)[None, :]
    flat2_bf = (2.0 * flat).astype(jnp.bfloat16)
    e_bf = e.astype(jnp.bfloat16)

    idx3, dsum = pl.pallas_call(
        _dist_argmin_body,
        grid=(N_TOKENS // BR,),
        in_specs=[
            pl.BlockSpec((BR, D), lambda i: (i, 0)),
            pl.BlockSpec((BR, 1), lambda i: (i, 0)),
            pl.BlockSpec((N_CODES, D), lambda i: (0, 0)),
            pl.BlockSpec((1, N_CODES), lambda i: (0, 0)),
            pl.BlockSpec((1, N_CODES), lambda i: (0, 0)),
        ],
        out_specs=[
            pl.BlockSpec((1, 1, BR), lambda i: (i, 0, 0)),
            pl.BlockSpec((1, 128), lambda i: (0, 0)),
        ],
        out_shape=[
            jax.ShapeDtypeStruct((N_TOKENS // BR, 1, BR), jnp.int32),
            jax.ShapeDtypeStruct((1, 128), jnp.float32),
        ],
    )(flat2_bf, srow, e_bf, se,
      jnp.arange(N_CODES, dtype=jnp.float32)[None, :])

    indices = idx3.reshape(N_TOKENS)
    quantized = _make_sc_gather()(e_bf.astype(jnp.float32), indices)

    mse = dsum[0, 0] / (N_TOKENS * D)
    vq_loss = mse + COMMIT * mse

    q5 = quantized.reshape(B, T, H, W, D)
    quantized_st = x_flat + lax.stop_gradient(q5 - x_flat)
    quantized_st = jnp.transpose(quantized_st, (0, 4, 1, 2, 3)).astype(x_dtype)
    indices_out = indices.reshape(B, T, H, W)
    return quantized_st, vq_loss, indices_out


# speedup vs baseline: 1.3052x; 1.0201x over previous
import functools

import jax
import jax.numpy as jnp
from jax import lax
from jax.experimental import pallas as pl
from jax.experimental.pallas import tpu as pltpu
from jax.experimental.pallas import tpu_sc as plsc

N_CODES = 8192
EMB_D = 256
N_TOKENS = 8192
BR = 2048
COMMIT = 0.25

NW = 32
ROWS_PER_W = N_TOKENS // NW
GCH = 128


CH = 1024


def _dist_argmin_body(flat2_ref, srow_ref, e_ref, se_ref, fio_ref,
                      idx_ref, dsum_ref):
    i = pl.program_id(0)
    flat2 = flat2_ref[...]
    srow = srow_ref[...]
    big = float(2 * N_CODES)
    HC = N_CODES // 2
    half_v, half_f = [], []
    for h in range(2):
        rv = rf = None
        for c in range(HC // CH):
            j0 = h * HC + c * CH
            e_c = e_ref[pl.ds(j0, CH), :]
            m2 = lax.dot_general(flat2, e_c, (((1,), (1,)), ((), ())),
                                 preferred_element_type=jnp.float32)
            dist = (srow + se_ref[:, pl.ds(j0, CH)]) - m2
            cv = jnp.min(dist, axis=1, keepdims=True)
            cf = jnp.min(jnp.where(dist == cv, fio_ref[:, pl.ds(j0, CH)],
                                   big), axis=1, keepdims=True)
            if rv is None:
                rv, rf = cv, cf
            else:
                t = cv < rv
                rv = jnp.where(t, cv, rv)
                rf = jnp.where(t, cf, rf)
        half_v.append(rv)
        half_f.append(rf)
    v1b = half_v[0].astype(jnp.bfloat16).astype(jnp.float32)
    take1 = v1b <= half_v[1]
    idx = jnp.where(take1[:, 0], half_f[0][:, 0],
                    half_f[1][:, 0]).astype(jnp.int32)
    idx_ref[0, 0, :] = idx
    minval = jnp.where(take1, half_v[0], half_v[1])
    bsum = jnp.full((1, 128), jnp.sum(minval), jnp.float32)

    @pl.when(i == 0)
    def _():
        dsum_ref[...] = jnp.zeros((1, 128), jnp.float32)

    dsum_ref[...] += bsum


@functools.lru_cache(maxsize=1)
def _make_sc_gather():
    mesh = plsc.VectorSubcoreMesh(core_axis_name="c", subcore_axis_name="s",
                                  num_cores=2, num_subcores=16)

    @functools.partial(
        pl.kernel,
        out_type=jax.ShapeDtypeStruct((N_TOKENS, EMB_D), jnp.float32),
        mesh=mesh,
        scratch_types=[
            pltpu.VMEM((GCH,), jnp.int32),
            pltpu.VMEM((GCH,), jnp.int32),
            pltpu.VMEM((GCH, EMB_D), jnp.float32),
            pltpu.VMEM((GCH, EMB_D), jnp.float32),
            pltpu.SemaphoreType.DMA,
            pltpu.SemaphoreType.DMA,
        ],
    )
    def _sc_gather(table_hbm, idx_hbm, out_hbm, idx_v0, idx_v1,
                   rows_v0, rows_v1, sem0, sem1):
        wid = lax.axis_index("s") * 2 + lax.axis_index("c")
        base = wid * ROWS_PER_W
        idx_vs = (idx_v0, idx_v1)
        rows_vs = (rows_v0, rows_v1)
        sems = (sem0, sem1)
        n_ch = ROWS_PER_W // GCH
        for j in range(n_ch):
            pltpu.sync_copy(idx_hbm.at[pl.ds(base + j * GCH, GCH)],
                            idx_vs[j % 2])
            pltpu.async_copy(table_hbm.at[idx_vs[j % 2]], rows_vs[j % 2],
                             sems[j % 2])
        for j in range(n_ch):
            pltpu.make_async_copy(table_hbm.at[idx_vs[j % 2]], rows_vs[j % 2],
                                  sems[j % 2]).wait()
            pltpu.sync_copy(rows_vs[j % 2],
                            out_hbm.at[pl.ds(base + j * GCH, GCH)])

    return _sc_gather


def kernel(x, emb):
    B, D, T, H, W = x.shape
    x_dtype = x.dtype
    x32 = x.astype(jnp.float32)
    x_flat = jnp.transpose(x32, (0, 2, 3, 4, 1))
    flat = x_flat.reshape(-1, D)
    e = emb.astype(jnp.float32)
    srow = jnp.sum(flat ** 2, axis=1, keepdims=True)
    se = jnp.sum(e ** 2, axis=1)[None, :]
    flat2_bf = (2.0 * flat).astype(jnp.bfloat16)
    e_bf = e.astype(jnp.bfloat16)

    idx3, dsum = pl.pallas_call(
        _dist_argmin_body,
        grid=(N_TOKENS // BR,),
        in_specs=[
            pl.BlockSpec((BR, D), lambda i: (i, 0)),
            pl.BlockSpec((BR, 1), lambda i: (i, 0)),
            pl.BlockSpec((N_CODES, D), lambda i: (0, 0)),
            pl.BlockSpec((1, N_CODES), lambda i: (0, 0)),
            pl.BlockSpec((1, N_CODES), lambda i: (0, 0)),
        ],
        out_specs=[
            pl.BlockSpec((1, 1, BR), lambda i: (i, 0, 0)),
            pl.BlockSpec((1, 128), lambda i: (0, 0)),
        ],
        out_shape=[
            jax.ShapeDtypeStruct((N_TOKENS // BR, 1, BR), jnp.int32),
            jax.ShapeDtypeStruct((1, 128), jnp.float32),
        ],
    )(flat2_bf, srow, e_bf, se,
      jnp.arange(N_CODES, dtype=jnp.float32)[None, :])

    indices = idx3.reshape(N_TOKENS)
    quantized = _make_sc_gather()(e_bf.astype(jnp.float32), indices)

    mse = dsum[0, 0] / (N_TOKENS * D)
    vq_loss = mse + COMMIT * mse

    q5 = quantized.reshape(B, T, H, W, D)
    quantized_st = x_flat + lax.stop_gradient(q5 - x_flat)
    quantized_st = jnp.transpose(quantized_st, (0, 4, 1, 2, 3)).astype(x_dtype)
    indices_out = indices.reshape(B, T, H, W)
    return quantized_st, vq_loss, indices_out
